# baseline (speedup 1.0000x reference)
import functools

import jax
import jax.numpy as jnp
from jax import lax
from jax.experimental import pallas as pl
from jax.experimental.pallas import tpu as pltpu
from jax.experimental.pallas import tpu_sc as plsc

_NC = 2
_NS = 16
_NW = _NC * _NS
_LANES = 16
_PAD = 129


def kernel(x, token_table, pos_table):
    B, T = x.shape
    V, E = token_table.shape
    BB = B // 128
    TC = 2
    n_chunks = T // TC
    half = n_chunks // 2
    R = TC * 128
    RP = R + 8

    big_table = jnp.concatenate([token_table, pos_table], axis=0)

    xtok = (x.T.astype(jnp.int32)
            .reshape(n_chunks, TC, BB, 128)
            .transpose(0, 2, 1, 3)
            .reshape(n_chunks, BB, R))
    pids = V + jnp.arange(T, dtype=jnp.int32).reshape(n_chunks, TC)
    pids = jnp.pad(pids, ((0, 0), (0, 8 - TC)))
    pids = jnp.broadcast_to(pids[:, None, :], (n_chunks, BB, 8))
    xt = jnp.concatenate([xtok, pids], axis=2)

    mesh = plsc.VectorSubcoreMesh(core_axis_name="c", subcore_axis_name="s")

    @functools.partial(
        pl.kernel,
        mesh=mesh,
        compiler_params=pltpu.CompilerParams(
            use_tc_tiling_on_sc=False, needs_layout_passes=False),
        out_type=jax.ShapeDtypeStruct((T, E // 8, BB, 8, 128), jnp.float32),
        scratch_types=[
            pltpu.VMEM((2, RP), jnp.int32),
            pltpu.VMEM((2, RP, E), jnp.float32),
            pltpu.VMEM((2, TC, E // 8, 8, _PAD), jnp.float32),
            pltpu.SemaphoreType.DMA,
            pltpu.SemaphoreType.DMA,
            pltpu.SemaphoreType.DMA,
            pltpu.SemaphoreType.DMA,
            pltpu.SemaphoreType.DMA,
            pltpu.SemaphoreType.DMA,
        ],
    )
    def emb_kernel(xt_hbm, tab_hbm, out_hbm,
                   idx_v, rows_v, obuf, g0, g1, is0, is1, ss0, ss1):
        gsem = (g0, g1)
        isem = (is0, is1)
        ssem = (ss0, ss1)
        wid = lax.axis_index("s") * _NC + lax.axis_index("c")

        def idx_copy(i, b):
            return pltpu.make_async_copy(
                xt_hbm.at[i, wid], idx_v.at[b], isem[b])

        def gather_copy(b):
            return pltpu.make_async_copy(
                tab_hbm.at[idx_v.at[b]], rows_v.at[b], gsem[b])

        def store_copy(i, b):
            return pltpu.make_async_copy(
                obuf.at[b, :, :, :, pl.ds(0, 128)],
                out_hbm.at[pl.ds(i * TC, TC), :, wid], ssem[b])

        idx_copy(0, 0).start()
        idx_copy(0, 0).wait()
        gather_copy(0).start()
        idx_copy(1, 1).start()

        def process(i, b):
            @pl.when(i + 1 < n_chunks)
            def _():
                idx_copy(i + 1, 1 - b).wait()
                gather_copy(1 - b).start()

            gather_copy(b).wait()

            @pl.when(i + 2 < n_chunks)
            def _():
                idx_copy(i + 2, b).start()

            @pl.when(i >= 2)
            def _():
                store_copy(i - 2, b).wait()

            pos_regs = [
                [rows_v[b, R + tl, pl.ds(c * _LANES, _LANES)]
                 for c in range(E // _LANES)]
                for tl in range(TC)
            ]

            def r_body(r2, carry):
                iota = lax.iota(jnp.int32, _LANES)
                for dr in range(2):
                    r = r2 * 2 + dr
                    rsp = jnp.zeros((_LANES,), jnp.int32) + r
                    vals = [
                        rows_v[b, tl * 128 + r, pl.ds(c * _LANES, _LANES)]
                        + pos_regs[tl][c]
                        for tl in range(TC) for c in range(E // _LANES)
                    ]
                    k = 0
                    for tl in range(TC):
                        for c in range(E // _LANES):
                            ev = iota + c * _LANES
                            plsc.store_scatter(
                                obuf,
                                [jnp.full((_LANES,), b, dtype=jnp.int32),
                                 jnp.full((_LANES,), tl, dtype=jnp.int32),
                                 ev >> 3, ev & 7, rsp],
                                vals[k])
                            k += 1
                return carry

            lax.fori_loop(0, 64, r_body, 0)
            store_copy(i, b).start()

        def pair_body(h, carry):
            process(2 * h, 0)
            process(2 * h + 1, 1)
            return carry

        lax.fori_loop(0, half, pair_body, 0)
        store_copy(n_chunks - 2, 0).wait()
        store_copy(n_chunks - 1, 1).wait()

    a = emb_kernel(xt, big_table)
    return a.transpose(2, 4, 0, 1, 3).reshape(B, T, E)


# speedup vs baseline: 2.1766x; 2.1766x over previous
import functools

import jax
import jax.numpy as jnp
from jax import lax
from jax.experimental import pallas as pl
from jax.experimental.pallas import tpu as pltpu
from jax.experimental.pallas import tpu_sc as plsc

_NC = 2
_NS = 16
_NW = _NC * _NS
_LANES = 16
_PAD = 129


def kernel(x, token_table, pos_table):
    B, T = x.shape
    V, E = token_table.shape
    BB = B // 128
    TC = 4
    n_chunks = T // TC
    half = n_chunks // 2
    R = TC * 128

    xt = (x.T.astype(jnp.int32)
          .reshape(n_chunks, TC, BB, 128)
          .transpose(0, 2, 1, 3)
          .reshape(n_chunks, BB, R))

    mesh = plsc.VectorSubcoreMesh(core_axis_name="c", subcore_axis_name="s")

    @functools.partial(
        pl.kernel,
        mesh=mesh,
        compiler_params=pltpu.CompilerParams(
            use_tc_tiling_on_sc=False, needs_layout_passes=False),
        out_type=jax.ShapeDtypeStruct((T, E // 8, BB, 8, 128), jnp.float32),
        scratch_types=[
            pltpu.VMEM((T, E), jnp.float32),
            pltpu.VMEM((2, R), jnp.int32),
            pltpu.VMEM((2, R, E), jnp.float32),
            pltpu.VMEM((2, TC // 2, E // 8, 8, _PAD), jnp.float32),
            pltpu.SemaphoreType.DMA,
            pltpu.SemaphoreType.DMA,
            pltpu.SemaphoreType.DMA,
            pltpu.SemaphoreType.DMA,
            pltpu.SemaphoreType.DMA,
            pltpu.SemaphoreType.DMA,
        ],
    )
    def emb_kernel(xt_hbm, tok_hbm, pos_hbm, out_hbm,
                   pos_v, idx_v, rows_v, obuf, g0, g1, is0, is1, ss0, ss1):
        gsem = (g0, g1)
        isem = (is0, is1)
        ssem = (ss0, ss1)
        wid = lax.axis_index("s") * _NC + lax.axis_index("c")

        def idx_copy(i, b):
            return pltpu.make_async_copy(
                xt_hbm.at[i, wid], idx_v.at[b], isem[b])

        def gather_copy(b):
            return pltpu.make_async_copy(
                tok_hbm.at[idx_v.at[b]], rows_v.at[b], gsem[b])

        def store_copy(i, h2):
            return pltpu.make_async_copy(
                obuf.at[h2, :, :, :, pl.ds(0, 128)],
                out_hbm.at[pl.ds(i * TC + h2 * (TC // 2), TC // 2), :, wid],
                ssem[h2])

        pltpu.sync_copy(pos_hbm, pos_v)
        idx_copy(0, 0).start()
        idx_copy(0, 0).wait()
        gather_copy(0).start()
        idx_copy(1, 1).start()

        def process(i, b):
            @pl.when(i + 1 < n_chunks)
            def _():
                idx_copy(i + 1, 1 - b).wait()
                gather_copy(1 - b).start()

            gather_copy(b).wait()

            @pl.when(i + 2 < n_chunks)
            def _():
                idx_copy(i + 2, b).start()

            pos_regs = [
                [pos_v[i * TC + tl, pl.ds(c * _LANES, _LANES)]
                 for c in range(E // _LANES)]
                for tl in range(TC)
            ]

            for h2 in range(2):
                @pl.when(i >= 1)
                def _(h2=h2):
                    store_copy(i - 1, h2).wait()

                def r_body(r2, carry, h2=h2):
                    iota = lax.iota(jnp.int32, _LANES)
                    for dr in range(2):
                        r = r2 * 2 + dr
                        rsp = jnp.zeros((_LANES,), jnp.int32) + r
                        tls = (h2 * 2, h2 * 2 + 1)
                        vals = [
                            rows_v[b, tl * 128 + r,
                                   pl.ds(c * _LANES, _LANES)]
                            + pos_regs[tl][c]
                            for tl in tls for c in range(E // _LANES)
                        ]
                        k = 0
                        for tj in range(2):
                            for c in range(E // _LANES):
                                ev = iota + c * _LANES
                                plsc.store_scatter(
                                    obuf,
                                    [jnp.full((_LANES,), h2,
                                              dtype=jnp.int32),
                                     jnp.full((_LANES,), tj,
                                              dtype=jnp.int32),
                                     ev >> 3, ev & 7, rsp],
                                    vals[k])
                                k += 1
                    return carry

                lax.fori_loop(0, 64, r_body, 0)
                store_copy(i, h2).start()

        def pair_body(h, carry):
            process(2 * h, 0)
            process(2 * h + 1, 1)
            return carry

        lax.fori_loop(0, half, pair_body, 0)
        store_copy(n_chunks - 1, 0).wait()
        store_copy(n_chunks - 1, 1).wait()

    a = emb_kernel(xt, token_table, pos_table)
    return a.transpose(2, 4, 0, 1, 3).reshape(B, T, E)
